# baseline (speedup 1.0000x reference)
import functools

import jax
import jax.numpy as jnp
from jax import lax
from jax.experimental import pallas as pl
from jax.experimental.pallas import tpu as pltpu
from jax.experimental.pallas import tpu_sc as plsc

BATCH = 16384
EMB = 64
BN_EPS = 1e-5

NC = 2
NS = 16
NW = NC * NS
BPW = BATCH // NW
CHUNK = 128
NCH = BPW // CHUNK

def _sc_gather4_body(uid, iid, ug_t, ig_t, um_t, im_t,
                     out_ug, out_ig, out_um, out_im,
                     uidx_v, iidx_v, buf_a, buf_b, gsem, osem):
    wid = lax.axis_index("s") * NC + lax.axis_index("c")
    base = wid * BPW

    pltpu.sync_copy(uid.at[pl.ds(wid * NCH, NCH)], uidx_v)
    pltpu.sync_copy(iid.at[pl.ds(wid * NCH, NCH)], iidx_v)

    jobs = (
        (ug_t, uidx_v, buf_a, out_ug),
        (ig_t, iidx_v, buf_b, out_ig),
        (um_t, uidx_v, buf_a, out_um),
        (im_t, iidx_v, buf_b, out_im),
    )

    gathers = []
    outs = [None, None, None, None]

    def fire(t):
        table, idx, buf, _ = jobs[t]
        descs = []
        for j in range(NCH):
            descs.append(pltpu.async_copy(
                table.at[idx.at[j]], buf.at[pl.ds(j * CHUNK, CHUNK)], gsem))
        gathers.append(descs)

    def drain_gather(t):
        for d in gathers[t]:
            d.wait()

    def writeback(t):
        _, _, buf, out = jobs[t]
        outs[t] = pltpu.async_copy(buf, out.at[pl.ds(base, BPW)], osem)

    fire(0)
    fire(1)
    drain_gather(0)
    writeback(0)
    drain_gather(1)
    writeback(1)
    outs[0].wait()
    fire(2)
    outs[1].wait()
    fire(3)
    drain_gather(2)
    writeback(2)
    drain_gather(3)
    writeback(3)
    outs[2].wait()
    outs[3].wait()


@functools.cache
def _make_sc_gather4():
    mesh = plsc.VectorSubcoreMesh(
        core_axis_name="c", subcore_axis_name="s",
        num_cores=NC, num_subcores=NS)
    return pl.kernel(
        _sc_gather4_body,
        out_type=(
            jax.ShapeDtypeStruct((BATCH, EMB), jnp.float32),
            jax.ShapeDtypeStruct((BATCH, EMB), jnp.float32),
            jax.ShapeDtypeStruct((BATCH, EMB), jnp.float32),
            jax.ShapeDtypeStruct((BATCH, EMB), jnp.float32),
        ),
        mesh=mesh,
        scratch_types=[
            pltpu.VMEM((NCH, CHUNK), jnp.int32),
            pltpu.VMEM((NCH, CHUNK), jnp.int32),
            pltpu.VMEM((BPW, EMB), jnp.float32),
            pltpu.VMEM((BPW, EMB), jnp.float32),
            pltpu.SemaphoreType.DMA,
            pltpu.SemaphoreType.DMA,
        ],
        compiler_params=pltpu.CompilerParams(use_tc_tiling_on_sc=False),
    )


BM = 2048


def _mlp_body(ug, ig, um, im, w1a, w1b, b1, w2, b2, w3, b3,
              wn1g, wn1h, bn1, wn2, bn2, out):
    f32 = jnp.float32
    y1 = jnp.maximum(
        jnp.dot(um[...], w1a[...], preferred_element_type=f32)
        + jnp.dot(im[...], w1b[...], preferred_element_type=f32)
        + b1[...], 0.0)
    y2 = jnp.maximum(
        jnp.dot(y1, w2[...], preferred_element_type=f32) + b2[...], 0.0)
    y3 = jnp.maximum(
        jnp.dot(y2, w3[...], preferred_element_type=f32) + b3[...], 0.0)
    g = ug[...] * ig[...]
    z = jnp.maximum(
        jnp.dot(g, wn1g[...], preferred_element_type=f32)
        + jnp.dot(y3, wn1h[...], preferred_element_type=f32)
        + bn1[...], 0.0)
    logit = jnp.dot(z, wn2[...], preferred_element_type=f32) + bn2[...]
    out[...] = jax.nn.sigmoid(logit)[:, 0]


_full = lambda shape: pl.BlockSpec(shape, lambda i: (0, 0))

_mlp_call = pl.pallas_call(
    _mlp_body,
    grid=(BATCH // BM,),
    in_specs=[
        pl.BlockSpec((BM, EMB), lambda i: (i, 0)),
        pl.BlockSpec((BM, EMB), lambda i: (i, 0)),
        pl.BlockSpec((BM, EMB), lambda i: (i, 0)),
        pl.BlockSpec((BM, EMB), lambda i: (i, 0)),
        _full((EMB, 128)),
        _full((EMB, 128)),
        _full((1, 128)),
        _full((128, 64)),
        _full((1, 64)),
        _full((64, 32)),
        _full((1, 32)),
        _full((EMB, 32)),
        _full((32, 32)),
        _full((1, 32)),
        _full((32, 1)),
        _full((1, 1)),
    ],
    out_specs=pl.BlockSpec((BM,), lambda i: (i,)),
    out_shape=jax.ShapeDtypeStruct((BATCH,), jnp.float32),
)


def kernel(user_ids, item_ids, user_gmf, item_gmf, user_mlp, item_mlp,
           W1, b1, g1, be1, W2, b2, g2, be2, W3, b3, g3, be3,
           Wn1, bn1, Wn2, bn2):
    inv = 1.0 / jnp.sqrt(jnp.float32(1.0) + BN_EPS)
    s1 = g1 * inv
    s2 = g2 * inv
    s3 = g3 * inv

    w1a = W1[:, :EMB].T
    w1b = W1[:, EMB:].T
    b1v = b1.reshape(1, -1)
    w2t = (W2 * s1[None, :]).T
    b2v = (b2 + W2 @ be1).reshape(1, -1)
    w3t = (W3 * s2[None, :]).T
    b3v = (b3 + W3 @ be2).reshape(1, -1)
    wn1g = Wn1[:, :EMB].T
    wn1h = (Wn1[:, EMB:] * s3[None, :]).T
    bn1v = (bn1 + Wn1[:, EMB:] @ be3).reshape(1, -1)
    wn2t = Wn2.T
    bn2v = bn2.reshape(1, 1)

    uid2d = user_ids.astype(jnp.int32).reshape(NW * NCH, CHUNK)
    iid2d = item_ids.astype(jnp.int32).reshape(NW * NCH, CHUNK)

    ug, ig, um, im = _make_sc_gather4()(uid2d, iid2d, user_gmf, item_gmf,
                                        user_mlp, item_mlp)

    return _mlp_call(ug, ig, um, im, w1a, w1b, b1v, w2t, b2v, w3t, b3v,
                     wn1g, wn1h, bn1v, wn2t, bn2v)


# speedup vs baseline: 1.5177x; 1.5177x over previous
import functools

import jax
import jax.numpy as jnp
from jax import lax
from jax.experimental import pallas as pl
from jax.experimental.pallas import tpu as pltpu
from jax.experimental.pallas import tpu_sc as plsc

BATCH = 16384
EMB = 64
BN_EPS = 1e-5

NC = 2
NS = 16
NW = NC * NS
BPW = BATCH // NW
TILE = 8
GRP = 16
NGRP = BPW // GRP
OTPW = BPW // TILE
HALF = BPW // 2


def _sc_gather4_body(uid, iid, ug_t, ig_t, um_t, im_t,
                     out_ug, out_ig, out_um, out_im,
                     usm, ism, buf, sem_a, sem_b, osem):
    wid = lax.axis_index("s") * NC + lax.axis_index("c")
    base = wid * BPW

    pltpu.sync_copy(uid.at[pl.ds(base, BPW)], usm)
    pltpu.sync_copy(iid.at[pl.ds(base, BPW)], ism)

    def fire_table(t2, idx_ref, half, slot, sem):
        def body(g, carry):
            off = pl.multiple_of(half * HALF + g * GRP, GRP)
            u = idx_ref[pl.ds(off, GRP)]
            for k in range(GRP):
                pltpu.async_copy(t2.at[pl.ds(u[k], 1)],
                                 buf.at[slot, pl.ds(g * GRP + k, 1)], sem)
            return carry
        lax.fori_loop(0, HALF // GRP, body, 0)

    def drain_table(t2, slot, sem):
        pltpu.make_async_copy(t2.at[pl.ds(0, HALF)],
                              buf.at[slot], sem).wait()

    def out_copy(half, slot, out):
        return pltpu.async_copy(
            buf.at[slot], out.at[pl.ds(base + half * HALF, HALF)], osem)

    units = [(t, idx, h)
             for h in (0, 1)
             for (t, idx) in ((ug_t, usm), (ig_t, ism), (um_t, usm), (im_t, ism))]
    outs = [out_ug, out_ig, out_um, out_im] * 2
    sems = [sem_a, sem_b]
    pending = [None, None]

    for n, (t2, idx_sm, h) in enumerate(units):
        slot = n % 2
        if pending[slot] is not None:
            pending[slot].wait()
        fire_table(t2, idx_sm, h, slot, sems[slot])
        drain_table(t2, slot, sems[slot])
        pending[slot] = out_copy(h, slot, outs[n])
    pending[0].wait()
    pending[1].wait()


@functools.cache
def _make_sc_gather4():
    mesh = plsc.VectorSubcoreMesh(
        core_axis_name="c", subcore_axis_name="s",
        num_cores=NC, num_subcores=NS)
    out = jax.ShapeDtypeStruct((BATCH, EMB), jnp.float32)
    return pl.kernel(
        _sc_gather4_body,
        out_type=(out, out, out, out),
        mesh=mesh,
        scratch_types=[
            pltpu.VMEM((BPW,), jnp.int32),
            pltpu.VMEM((BPW,), jnp.int32),
            pltpu.VMEM((2, HALF, EMB), jnp.float32),
            pltpu.SemaphoreType.DMA,
            pltpu.SemaphoreType.DMA,
            pltpu.SemaphoreType.DMA,
        ],
        compiler_params=pltpu.CompilerParams(needs_layout_passes=False),
    )


BM = 2048


def _mlp_body(ug, ig, um, im, w1a, w1b, b1, w2, b2, w3, b3,
              wn1g, wn1h, bn1, wn2, bn2, out):
    f32 = jnp.float32
    y1 = jnp.maximum(
        jnp.dot(um[...], w1a[...], preferred_element_type=f32)
        + jnp.dot(im[...], w1b[...], preferred_element_type=f32)
        + b1[...], 0.0)
    y2 = jnp.maximum(
        jnp.dot(y1, w2[...], preferred_element_type=f32) + b2[...], 0.0)
    y3 = jnp.maximum(
        jnp.dot(y2, w3[...], preferred_element_type=f32) + b3[...], 0.0)
    g = ug[...] * ig[...]
    z = jnp.maximum(
        jnp.dot(g, wn1g[...], preferred_element_type=f32)
        + jnp.dot(y3, wn1h[...], preferred_element_type=f32)
        + bn1[...], 0.0)
    logit = jnp.dot(z, wn2[...], preferred_element_type=f32) + bn2[...]
    out[...] = jax.nn.sigmoid(logit)[:, 0]


_full = lambda shape: pl.BlockSpec(shape, lambda i: (0, 0))

_mlp_call = pl.pallas_call(
    _mlp_body,
    grid=(BATCH // BM,),
    in_specs=[
        pl.BlockSpec((BM, EMB), lambda i: (i, 0)),
        pl.BlockSpec((BM, EMB), lambda i: (i, 0)),
        pl.BlockSpec((BM, EMB), lambda i: (i, 0)),
        pl.BlockSpec((BM, EMB), lambda i: (i, 0)),
        _full((EMB, 128)),
        _full((EMB, 128)),
        _full((1, 128)),
        _full((128, 64)),
        _full((1, 64)),
        _full((64, 32)),
        _full((1, 32)),
        _full((EMB, 32)),
        _full((32, 32)),
        _full((1, 32)),
        _full((32, 1)),
        _full((1, 1)),
    ],
    out_specs=pl.BlockSpec((BM,), lambda i: (i,)),
    out_shape=jax.ShapeDtypeStruct((BATCH,), jnp.float32),
)


def kernel(user_ids, item_ids, user_gmf, item_gmf, user_mlp, item_mlp,
           W1, b1, g1, be1, W2, b2, g2, be2, W3, b3, g3, be3,
           Wn1, bn1, Wn2, bn2):
    inv = 1.0 / jnp.sqrt(jnp.float32(1.0) + BN_EPS)
    s1 = g1 * inv
    s2 = g2 * inv
    s3 = g3 * inv

    w1a = W1[:, :EMB].T
    w1b = W1[:, EMB:].T
    b1v = b1.reshape(1, -1)
    w2t = (W2 * s1[None, :]).T
    b2v = (b2 + W2 @ be1).reshape(1, -1)
    w3t = (W3 * s2[None, :]).T
    b3v = (b3 + W3 @ be2).reshape(1, -1)
    wn1g = Wn1[:, :EMB].T
    wn1h = (Wn1[:, EMB:] * s3[None, :]).T
    bn1v = (bn1 + Wn1[:, EMB:] @ be3).reshape(1, -1)
    wn2t = Wn2.T
    bn2v = bn2.reshape(1, 1)

    uid = user_ids.astype(jnp.int32)
    iid = item_ids.astype(jnp.int32)

    ug, ig, um, im = _make_sc_gather4()(uid, iid, user_gmf, item_gmf,
                                        user_mlp, item_mlp)

    return _mlp_call(ug, ig, um, im, w1a, w1b, b1v, w2t, b2v, w3t, b3v,
                     wn1g, wn1h, bn1v, wn2t, bn2v)


# speedup vs baseline: 1.5208x; 1.0021x over previous
import functools

import jax
import jax.numpy as jnp
from jax import lax
from jax.experimental import pallas as pl
from jax.experimental.pallas import tpu as pltpu
from jax.experimental.pallas import tpu_sc as plsc

BATCH = 16384
EMB = 64
BN_EPS = 1e-5

NC = 2
NS = 16
NW = NC * NS
BPW = BATCH // NW
TILE = 8
GRP = 16
NGRP = BPW // GRP
OTPW = BPW // TILE
HALF = BPW // 2


def _sc_gather4_body(uid, iid, ug_t, ig_t, um_t, im_t,
                     out_ug, out_ig, out_um, out_im,
                     usm, ism, buf, sem_a, sem_b, osem):
    wid = lax.axis_index("s") * NC + lax.axis_index("c")
    base = wid * BPW

    pltpu.sync_copy(uid.at[pl.ds(base, BPW)], usm)
    pltpu.sync_copy(iid.at[pl.ds(base, BPW)], ism)

    def fire_table(t2, idx_ref, half, slot, sem):
        def body(g, carry):
            off = pl.multiple_of(half * HALF + g * GRP, GRP)
            u = idx_ref[pl.ds(off, GRP)]
            for k in range(GRP):
                pltpu.async_copy(t2.at[pl.ds(u[k], 1)],
                                 buf.at[slot, pl.ds(g * GRP + k, 1)], sem)
            return carry
        lax.fori_loop(0, HALF // GRP, body, 0)

    def drain_table(t2, slot, sem):
        pltpu.make_async_copy(t2.at[pl.ds(0, HALF)],
                              buf.at[slot], sem).wait()

    def out_copy(half, slot, out):
        return pltpu.async_copy(
            buf.at[slot], out.at[pl.ds(base + half * HALF, HALF)], osem)

    units = [(t, idx, h)
             for h in (0, 1)
             for (t, idx) in ((ug_t, usm), (ig_t, ism), (um_t, usm), (im_t, ism))]
    outs = [out_ug, out_ig, out_um, out_im] * 2
    sems = [sem_a, sem_b]
    pending = [None, None]

    for n, (t2, idx_sm, h) in enumerate(units):
        slot = n % 2
        if pending[slot] is not None:
            pending[slot].wait()
        fire_table(t2, idx_sm, h, slot, sems[slot])
        drain_table(t2, slot, sems[slot])
        pending[slot] = out_copy(h, slot, outs[n])
    pending[0].wait()
    pending[1].wait()


@functools.cache
def _make_sc_gather4():
    mesh = plsc.VectorSubcoreMesh(
        core_axis_name="c", subcore_axis_name="s",
        num_cores=NC, num_subcores=NS)
    out = jax.ShapeDtypeStruct((BATCH, EMB), jnp.float32)
    return pl.kernel(
        _sc_gather4_body,
        out_type=(out, out, out, out),
        mesh=mesh,
        scratch_types=[
            pltpu.VMEM((BPW,), jnp.int32),
            pltpu.VMEM((BPW,), jnp.int32),
            pltpu.VMEM((2, HALF, EMB), jnp.float32),
            pltpu.SemaphoreType.DMA,
            pltpu.SemaphoreType.DMA,
            pltpu.SemaphoreType.DMA,
        ],
    )


BM = 2048


def _mlp_body(ug, ig, um, im, w1a, w1b, b1, w2, b2, w3, b3,
              wn1g, wn1h, bn1, wn2, bn2, out):
    f32 = jnp.float32
    y1 = jnp.maximum(
        jnp.dot(um[...], w1a[...], preferred_element_type=f32)
        + jnp.dot(im[...], w1b[...], preferred_element_type=f32)
        + b1[...], 0.0)
    y2 = jnp.maximum(
        jnp.dot(y1, w2[...], preferred_element_type=f32) + b2[...], 0.0)
    y3 = jnp.maximum(
        jnp.dot(y2, w3[...], preferred_element_type=f32) + b3[...], 0.0)
    g = ug[...] * ig[...]
    z = jnp.maximum(
        jnp.dot(g, wn1g[...], preferred_element_type=f32)
        + jnp.dot(y3, wn1h[...], preferred_element_type=f32)
        + bn1[...], 0.0)
    logit = jnp.dot(z, wn2[...], preferred_element_type=f32) + bn2[...]
    out[...] = jax.nn.sigmoid(logit)[:, 0]


_full = lambda shape: pl.BlockSpec(shape, lambda i: (0, 0))

_mlp_call = pl.pallas_call(
    _mlp_body,
    grid=(BATCH // BM,),
    in_specs=[
        pl.BlockSpec((BM, EMB), lambda i: (i, 0)),
        pl.BlockSpec((BM, EMB), lambda i: (i, 0)),
        pl.BlockSpec((BM, EMB), lambda i: (i, 0)),
        pl.BlockSpec((BM, EMB), lambda i: (i, 0)),
        _full((EMB, 128)),
        _full((EMB, 128)),
        _full((1, 128)),
        _full((128, 64)),
        _full((1, 64)),
        _full((64, 32)),
        _full((1, 32)),
        _full((EMB, 32)),
        _full((32, 32)),
        _full((1, 32)),
        _full((32, 1)),
        _full((1, 1)),
    ],
    out_specs=pl.BlockSpec((BM,), lambda i: (i,)),
    out_shape=jax.ShapeDtypeStruct((BATCH,), jnp.float32),
)


def kernel(user_ids, item_ids, user_gmf, item_gmf, user_mlp, item_mlp,
           W1, b1, g1, be1, W2, b2, g2, be2, W3, b3, g3, be3,
           Wn1, bn1, Wn2, bn2):
    inv = 1.0 / jnp.sqrt(jnp.float32(1.0) + BN_EPS)
    s1 = g1 * inv
    s2 = g2 * inv
    s3 = g3 * inv

    w1a = W1[:, :EMB].T
    w1b = W1[:, EMB:].T
    b1v = b1.reshape(1, -1)
    w2t = (W2 * s1[None, :]).T
    b2v = (b2 + W2 @ be1).reshape(1, -1)
    w3t = (W3 * s2[None, :]).T
    b3v = (b3 + W3 @ be2).reshape(1, -1)
    wn1g = Wn1[:, :EMB].T
    wn1h = (Wn1[:, EMB:] * s3[None, :]).T
    bn1v = (bn1 + Wn1[:, EMB:] @ be3).reshape(1, -1)
    wn2t = Wn2.T
    bn2v = bn2.reshape(1, 1)

    uid = user_ids.astype(jnp.int32)
    iid = item_ids.astype(jnp.int32)

    ug, ig, um, im = _make_sc_gather4()(uid, iid, user_gmf, item_gmf,
                                        user_mlp, item_mlp)

    return _mlp_call(ug, ig, um, im, w1a, w1b, b1v, w2t, b2v, w3t, b3v,
                     wn1g, wn1h, bn1v, wn2t, bn2v)
